# baseline (speedup 1.0000x reference)
import jax
import jax.numpy as jnp
from jax.experimental import pallas as pl
from jax.experimental.pallas import tpu as pltpu

TEMP = 8.0
LOG2E = 1.4426950408889634
N = 4096
NTRK = 512
D = 32
R = 1024
ACC_ROWS = 8


def _loss_kernel(xb_ref, y16_ref, x16_ref, trk_ref, out_ref, acc_ref):
    i = pl.program_id(0)
    nsteps = pl.num_programs(0)

    @pl.when(i == 0)
    def _init():
        acc_ref[...] = jnp.zeros_like(acc_ref)

    xb = xb_ref[...] * (LOG2E / TEMP)
    xb16 = xb.astype(jnp.bfloat16)

    dot = lambda a, b: jax.lax.dot_general(
        a, b, (((1,), (1,)), ((), ())), preferred_element_type=jnp.float32)
    s_y = jnp.exp2(dot(xb16, y16_ref[...]))
    s_x = jnp.exp2(dot(xb16, x16_ref[...]))

    t_blk = trk_ref[pl.ds(i * R, R)]
    t_col = t_blk[:, None]
    mask_x = trk_ref[...][None, :] == t_col

    F = s_y[:, 0:NTRK]
    for q in range(1, N // NTRK):
        F = F + s_y[:, q * NTRK:(q + 1) * NTRK]

    t_y = jnp.sum(F, axis=1)
    t_x = jnp.sum(s_x, axis=1)
    m_x = jnp.sum(jnp.where(mask_x, s_x, 0.0), axis=1)

    sumsq = jnp.sum(xb * xb, axis=1)
    self_full = jnp.exp2((TEMP / LOG2E) * sumsq)

    hot = (t_col == jax.lax.broadcasted_iota(jnp.int32, (R, NTRK), 1)
           ).astype(jnp.float32)
    cf = jnp.sum(hot * F, axis=0)

    a_num = 0.5 * (m_x - self_full)
    a_den = t_y + t_x - m_x
    vals = jnp.concatenate(
        [a_num[None, :], a_den[None, :], jnp.ones((1, R), jnp.float32),
         jnp.zeros((ACC_ROWS - 3, R), jnp.float32)], axis=0)
    contrib = jax.lax.dot_general(
        vals, hot, (((1,), (0,)), ((), ())),
        preferred_element_type=jnp.float32)
    ridx = jax.lax.broadcasted_iota(jnp.int32, (ACC_ROWS, 1), 0)
    sgn = jnp.where(ridx == 0, 1.0,
                    jnp.where(ridx == 1, -1.0, 0.0))
    acc_ref[...] += contrib + sgn * cf[None, :]

    @pl.when(i == nsteps - 1)
    def _finish():
        num = acc_ref[0, :]
        den = acc_ref[1, :]
        present = acc_ref[2, :] > 0.0
        safe_num = jnp.where(present, num, 1.0)
        safe_den = jnp.where(present, den, 1.0)
        per = jnp.where(
            present, -jnp.log(safe_num / (safe_den + safe_num)), 0.0)
        n_present = jnp.maximum(jnp.sum(present.astype(jnp.float32)), 1.0)
        out_ref[...] = (jnp.sum(per) / n_present).reshape(1)


@jax.jit
def _run(x, y16, x16, trk):
    nsteps = N // R
    return pl.pallas_call(
        _loss_kernel,
        grid=(nsteps,),
        in_specs=[
            pl.BlockSpec((R, D), lambda i: (i, 0)),
            pl.BlockSpec((N, D), lambda i: (0, 0)),
            pl.BlockSpec((N, D), lambda i: (0, 0)),
            pl.BlockSpec((N,), lambda i: (0,)),
        ],
        out_specs=pl.BlockSpec((1,), lambda i: (0,)),
        out_shape=jax.ShapeDtypeStruct((1,), jnp.float32),
        scratch_shapes=[pltpu.VMEM((ACC_ROWS, NTRK), jnp.float32)],
    )(x, y16, x16, trk)


def kernel(x, track_idxs, y):
    yf = y.reshape(-1, D)
    return _run(x, yf.astype(jnp.bfloat16), x.astype(jnp.bfloat16),
                track_idxs.astype(jnp.int32))


# speedup vs baseline: 1.0061x; 1.0061x over previous
import jax
import jax.numpy as jnp
from jax.experimental import pallas as pl
from jax.experimental.pallas import tpu as pltpu

TEMP = 8.0
LOG2E = 1.4426950408889634
N = 4096
NTRK = 512
D = 32
R = 1024
ACC_ROWS = 8


def _loss_kernel(xb_ref, comb_ref, trk_ref, out_ref, acc_ref):
    i = pl.program_id(0)
    nsteps = pl.num_programs(0)

    @pl.when(i == 0)
    def _init():
        acc_ref[...] = jnp.zeros_like(acc_ref)

    xb = xb_ref[...] * (LOG2E / TEMP)
    xb16 = xb.astype(jnp.bfloat16)

    Sc = jnp.exp2(jax.lax.dot_general(
        xb16, comb_ref[...], (((1,), (1,)), ((), ())),
        preferred_element_type=jnp.float32))
    s_y = Sc[:, :N]
    s_x = Sc[:, N:]

    t_blk = trk_ref[pl.ds(i * R, R)]
    t_col = t_blk[:, None]
    mask_x = trk_ref[...][None, :] == t_col

    F = s_y[:, 0:NTRK]
    for q in range(1, N // NTRK):
        F = F + s_y[:, q * NTRK:(q + 1) * NTRK]

    t_y = jnp.sum(F, axis=1)
    t_x = jnp.sum(s_x, axis=1)
    m_x = jnp.sum(jnp.where(mask_x, s_x, 0.0), axis=1)

    sumsq = jnp.sum(xb * xb, axis=1)
    self_full = jnp.exp2((TEMP / LOG2E) * sumsq)

    hot = (t_col == jax.lax.broadcasted_iota(jnp.int32, (R, NTRK), 1)
           ).astype(jnp.float32)
    cf = jnp.sum(hot * F, axis=0)

    a_num = 0.5 * (m_x - self_full)
    a_den = t_y + t_x - m_x
    vals = jnp.concatenate(
        [a_num[None, :], a_den[None, :], jnp.ones((1, R), jnp.float32),
         jnp.zeros((ACC_ROWS - 3, R), jnp.float32)], axis=0)
    contrib = jax.lax.dot_general(
        vals, hot, (((1,), (0,)), ((), ())),
        preferred_element_type=jnp.float32)
    ridx = jax.lax.broadcasted_iota(jnp.int32, (ACC_ROWS, 1), 0)
    sgn = jnp.where(ridx == 0, 1.0,
                    jnp.where(ridx == 1, -1.0, 0.0))
    acc_ref[...] += contrib + sgn * cf[None, :]

    @pl.when(i == nsteps - 1)
    def _finish():
        num = acc_ref[0, :]
        den = acc_ref[1, :]
        present = acc_ref[2, :] > 0.0
        safe_num = jnp.where(present, num, 1.0)
        safe_den = jnp.where(present, den, 1.0)
        per = jnp.where(
            present, -jnp.log(safe_num / (safe_den + safe_num)), 0.0)
        n_present = jnp.maximum(jnp.sum(present.astype(jnp.float32)), 1.0)
        out_ref[...] = (jnp.sum(per) / n_present).reshape(1)


@jax.jit
def _run(x, comb16, trk):
    nsteps = N // R
    return pl.pallas_call(
        _loss_kernel,
        grid=(nsteps,),
        in_specs=[
            pl.BlockSpec((R, D), lambda i: (i, 0)),
            pl.BlockSpec((2 * N, D), lambda i: (0, 0)),
            pl.BlockSpec((N,), lambda i: (0,)),
        ],
        out_specs=pl.BlockSpec((1,), lambda i: (0,)),
        out_shape=jax.ShapeDtypeStruct((1,), jnp.float32),
        scratch_shapes=[pltpu.VMEM((ACC_ROWS, NTRK), jnp.float32)],
    )(x, comb16, trk)


def kernel(x, track_idxs, y):
    comb16 = jnp.concatenate(
        [y.reshape(-1, D), x], axis=0).astype(jnp.bfloat16)
    return _run(x, comb16, track_idxs.astype(jnp.int32))


# speedup vs baseline: 1.0500x; 1.0436x over previous
import jax
import jax.numpy as jnp
from jax.experimental import pallas as pl
from jax.experimental.pallas import tpu as pltpu

TEMP = 8.0
LOG2E = 1.4426950408889634
N = 4096
NTRK = 512
D = 32
R = 1024
ACC_ROWS = 8


def _loss_kernel(xb_ref, xf_ref, y3_ref, trk_ref, out_ref, acc_ref, comb_ref):
    i = pl.program_id(0)
    nsteps = pl.num_programs(0)

    @pl.when(i == 0)
    def _init():
        acc_ref[...] = jnp.zeros_like(acc_ref)
        comb_ref[:N, :] = y3_ref[...].reshape(N, D).astype(jnp.bfloat16)
        comb_ref[N:, :] = xf_ref[...].astype(jnp.bfloat16)

    xb = xb_ref[...] * (LOG2E / TEMP)
    xb16 = xb.astype(jnp.bfloat16)

    Sc = jnp.exp2(jax.lax.dot_general(
        xb16, comb_ref[...], (((1,), (1,)), ((), ())),
        preferred_element_type=jnp.float32))
    s_y = Sc[:, :N]
    s_x = Sc[:, N:]

    t_blk = trk_ref[pl.ds(i * R, R)]
    t_col = t_blk[:, None]
    mask_x = trk_ref[...][None, :] == t_col

    F = s_y[:, 0:NTRK]
    for q in range(1, N // NTRK):
        F = F + s_y[:, q * NTRK:(q + 1) * NTRK]

    t_y = jnp.sum(F, axis=1)
    t_x = jnp.sum(s_x, axis=1)
    m_x = jnp.sum(jnp.where(mask_x, s_x, 0.0), axis=1)

    sumsq = jnp.sum(xb * xb, axis=1)
    self_full = jnp.exp2((TEMP / LOG2E) * sumsq)

    hot = (t_col == jax.lax.broadcasted_iota(jnp.int32, (R, NTRK), 1)
           ).astype(jnp.float32)
    cf = jnp.sum(hot * F, axis=0)

    a_num = 0.5 * (m_x - self_full)
    a_den = t_y + t_x - m_x
    vals = jnp.concatenate(
        [a_num[None, :], a_den[None, :], jnp.ones((1, R), jnp.float32),
         jnp.zeros((ACC_ROWS - 3, R), jnp.float32)], axis=0)
    contrib = jax.lax.dot_general(
        vals, hot, (((1,), (0,)), ((), ())),
        preferred_element_type=jnp.float32)
    ridx = jax.lax.broadcasted_iota(jnp.int32, (ACC_ROWS, 1), 0)
    sgn = jnp.where(ridx == 0, 1.0,
                    jnp.where(ridx == 1, -1.0, 0.0))
    acc_ref[...] += contrib + sgn * cf[None, :]

    @pl.when(i == nsteps - 1)
    def _finish():
        num = acc_ref[0, :]
        den = acc_ref[1, :]
        present = acc_ref[2, :] > 0.0
        safe_num = jnp.where(present, num, 1.0)
        safe_den = jnp.where(present, den, 1.0)
        per = jnp.where(
            present, -jnp.log(safe_num / (safe_den + safe_num)), 0.0)
        n_present = jnp.maximum(jnp.sum(present.astype(jnp.float32)), 1.0)
        out_ref[...] = (jnp.sum(per) / n_present).reshape(1)


@jax.jit
def _run(x, y, trk):
    nsteps = N // R
    return pl.pallas_call(
        _loss_kernel,
        grid=(nsteps,),
        in_specs=[
            pl.BlockSpec((R, D), lambda i: (i, 0)),
            pl.BlockSpec((N, D), lambda i: (0, 0)),
            pl.BlockSpec(y.shape, lambda i: (0, 0, 0)),
            pl.BlockSpec((N,), lambda i: (0,)),
        ],
        out_specs=pl.BlockSpec((1,), lambda i: (0,)),
        out_shape=jax.ShapeDtypeStruct((1,), jnp.float32),
        scratch_shapes=[pltpu.VMEM((ACC_ROWS, NTRK), jnp.float32),
                        pltpu.VMEM((2 * N, D), jnp.bfloat16)],
    )(x, x, y, trk)


def kernel(x, track_idxs, y):
    return _run(x, y, track_idxs.astype(jnp.int32))
